# baseline (speedup 1.0000x reference)
import dataclasses
import functools

import jax
import jax.numpy as jnp
import numpy as np
from jax import lax
from jax.experimental import pallas as pl
from jax.experimental.pallas import tpu as pltpu
from jax.experimental.pallas import tpu_sc as plsc

H, DM, DK, DV, DFF = 8, 128, 16, 16, 512
NV, NE = 10000, 320000
NB = 1000
EB = 2000
GW = 128
SW = 128
NPAD = 10240
NSC, NSUB = 2, 16
NR = NPAD * H // DM
CW = 2000

_f32 = jnp.float32


def _ln(y, g, b):
    m = jnp.mean(y, axis=-1, keepdims=True)
    v = jnp.mean((y - m) ** 2, axis=-1, keepdims=True)
    return (y - m) * jax.lax.rsqrt(v + 1e-5) * g + b


def _sa1_body(x_ref, wq_ref, wk_ref, wv_ref, q_ref, kv_ref, ks_ref,
              acc_kv, acc_ks):
    i = pl.program_id(0)

    @pl.when(i == 0)
    def _():
        acc_kv[...] = jnp.zeros_like(acc_kv)
        acc_ks[...] = jnp.zeros_like(acc_ks)

    x = x_ref[...]
    q = jax.nn.sigmoid(jnp.dot(x, wq_ref[...], preferred_element_type=_f32))
    k = jax.nn.sigmoid(jnp.dot(x, wk_ref[...], preferred_element_type=_f32))
    v = jnp.dot(x, wv_ref[...], preferred_element_type=_f32)
    q_ref[...] = q
    acc_kv[...] += lax.dot_general(k, v, (((0,), (0,)), ((), ())),
                                   preferred_element_type=_f32)
    acc_ks[...] += jnp.broadcast_to(jnp.sum(k, axis=0, keepdims=True), (8, DM))

    @pl.when(i == NV // NB - 1)
    def _():
        kv_ref[...] = acc_kv[...]
        ks_ref[...] = acc_ks[...]


def _sa2_body(q_ref, x_ref, kv_ref, ks_ref, fc_ref, lnp_ref, bd_ref, o_ref):
    q = q_ref[...]
    kvm = kv_ref[...] * bd_ref[...]
    num = jnp.dot(q, kvm, preferred_element_type=_f32)
    den = jnp.dot(q * ks_ref[0:1, :], bd_ref[...], preferred_element_type=_f32)
    out = num / (den + 1e-8)
    y = jnp.dot(out, fc_ref[...], preferred_element_type=_f32) + x_ref[...]
    o_ref[...] = _ln(y, lnp_ref[0:1, :], lnp_ref[1:2, :])


def _self_attn(x, p, bd):
    grid = (NV // NB,)
    row = lambda i: (i, 0)
    full = lambda i: (0, 0)
    q, kv, ks = pl.pallas_call(
        _sa1_body,
        grid=grid,
        in_specs=[pl.BlockSpec((NB, DM), row)] + [pl.BlockSpec((DM, DM), full)] * 3,
        out_specs=[pl.BlockSpec((NB, DM), row), pl.BlockSpec((DM, DM), full),
                   pl.BlockSpec((8, DM), full)],
        out_shape=[jax.ShapeDtypeStruct((NV, DM), _f32),
                   jax.ShapeDtypeStruct((DM, DM), _f32),
                   jax.ShapeDtypeStruct((8, DM), _f32)],
        scratch_shapes=[pltpu.VMEM((DM, DM), _f32), pltpu.VMEM((8, DM), _f32)],
    )(x, p['WQ'], p['WK'], p['WV'])
    lnp = jnp.stack([p['ln_g'], p['ln_b']])
    out = pl.pallas_call(
        _sa2_body,
        grid=grid,
        in_specs=[pl.BlockSpec((NB, DM), row), pl.BlockSpec((NB, DM), row),
                  pl.BlockSpec((DM, DM), full), pl.BlockSpec((8, DM), full),
                  pl.BlockSpec((DM, DM), full), pl.BlockSpec((2, DM), full),
                  pl.BlockSpec((DM, DM), full)],
        out_specs=pl.BlockSpec((NB, DM), row),
        out_shape=jax.ShapeDtypeStruct((NV, DM), _f32),
    )(q, x, kv, ks, p['fc'], lnp, bd)
    return out


def _projkvq_body(x_ref, wk_ref, wv_ref, wq_ref, kv_ref, q_ref):
    x = x_ref[...]
    kv_ref[:, 0:DM] = jnp.dot(x, wk_ref[...], preferred_element_type=_f32)
    kv_ref[:, DM:2 * DM] = jnp.dot(x, wv_ref[...], preferred_element_type=_f32)
    q_ref[...] = jnp.dot(x, wq_ref[...], preferred_element_type=_f32)


def _proj_kv_q(x, wk, wv, wq):
    row = lambda i: (i, 0)
    full = lambda i: (0, 0)
    return pl.pallas_call(
        _projkvq_body,
        grid=(NV // NB,),
        in_specs=[pl.BlockSpec((NB, DM), row)] + [pl.BlockSpec((DM, DM), full)] * 3,
        out_specs=[pl.BlockSpec((NB, 2 * DM), row), pl.BlockSpec((NB, DM), row)],
        out_shape=[jax.ShapeDtypeStruct((NV, 2 * DM), _f32),
                   jax.ShapeDtypeStruct((NV, DM), _f32)],
    )(x, wk, wv, wq)


def _proj2_body(x_ref, wa_ref, wb_ref, oa_ref, ob_ref):
    x = x_ref[...]
    oa_ref[...] = jnp.dot(x, wa_ref[...], preferred_element_type=_f32)
    ob_ref[...] = jnp.dot(x, wb_ref[...], preferred_element_type=_f32)


def _proj2(x, wa, wb):
    row = lambda i: (i, 0)
    full = lambda i: (0, 0)
    return pl.pallas_call(
        _proj2_body,
        grid=(NE // EB,),
        in_specs=[pl.BlockSpec((EB, DM), row)] + [pl.BlockSpec((DM, DM), full)] * 2,
        out_specs=[pl.BlockSpec((EB, DM), row)] * 2,
        out_shape=[jax.ShapeDtypeStruct((NE, DM), _f32)] * 2,
    )(x, wa, wb)


def _edge_body(q_ref, kv_ref, e_ref, bd_ref, sel_ref, m_ref, a_ref):
    q = q_ref[...]
    k = kv_ref[:, 0:DM]
    v = kv_ref[:, DM:2 * DM]
    s = q * k * e_ref[...]
    srep = jnp.dot(s, bd_ref[...], preferred_element_type=_f32) * 0.25
    attn = jnp.exp(jnp.clip(srep, -5.0, 5.0))
    m_ref[...] = attn * v
    a_ref[...] = jnp.dot(attn, sel_ref[...], preferred_element_type=_f32)


def _edge_math(qs, kvt, e, bd, sel):
    row = lambda i: (i, 0)
    full = lambda i: (0, 0)
    return pl.pallas_call(
        _edge_body,
        grid=(NE // EB,),
        in_specs=[pl.BlockSpec((EB, DM), row), pl.BlockSpec((EB, 2 * DM), row),
                  pl.BlockSpec((EB, DM), row), pl.BlockSpec((DM, DM), full),
                  pl.BlockSpec((DM, 8), full)],
        out_specs=[pl.BlockSpec((EB, DM), row), pl.BlockSpec((EB, 8), row)],
        out_shape=[jax.ShapeDtypeStruct((NE, DM), _f32),
                   jax.ShapeDtypeStruct((NE, 8), _f32)],
    )(qs, kvt, e, bd, sel)


def _sc_gather(qtab, kvtab, qidx, kvidx):
    mesh = plsc.VectorSubcoreMesh(core_axis_name="c", subcore_axis_name="s")
    W = 80
    npw = NE // (NSC * NSUB * W)

    @functools.partial(
        pl.kernel,
        out_type=[jax.ShapeDtypeStruct((NE, DM), _f32),
                  jax.ShapeDtypeStruct((NE, 2 * DM), _f32)],
        mesh=mesh,
        scratch_types=(
            [pltpu.VMEM((W,), jnp.int32)] * 4
            + [pltpu.VMEM((W, DM), _f32), pltpu.VMEM((W, 2 * DM), _f32)] * 2
            + [pltpu.SemaphoreType.DMA] * 6
        ),
    )
    def gk(q_hbm, kv_hbm, qi_hbm, ki_hbm, oq_hbm, okv_hbm,
           qi0, ki0, qi1, ki1, bq0, bkv0, bq1, bkv1,
           si0, si1, sg0, sg1, so0, so1):
        c = lax.axis_index("c")
        s = lax.axis_index("s")
        wid = s * NSC + c
        base = wid * npw * W
        bufs = ((qi0, ki0, bq0, bkv0, si0, sg0, so0),
                (qi1, ki1, bq1, bkv1, si1, sg1, so1))

        for b in (0, 1):
            off = base + b * W
            pltpu.async_copy(qi_hbm.at[pl.ds(off, W)], bufs[b][0], bufs[b][4])
            pltpu.async_copy(ki_hbm.at[pl.ds(off, W)], bufs[b][1], bufs[b][4])

        @pl.loop(0, npw, step=2)
        def _(j):
            for b in (0, 1):
                qi_v, ki_v, bq, bkv, si, sg, so = bufs[b]
                jj = j + b

                @pl.when(jj < npw)
                def _():
                    off = base + jj * W
                    pltpu.make_async_copy(qi_hbm.at[pl.ds(off, W)], qi_v,
                                          si).wait()
                    pltpu.make_async_copy(ki_hbm.at[pl.ds(off, W)], ki_v,
                                          si).wait()

                    @pl.when(jj >= 2)
                    def _():
                        pltpu.make_async_copy(bq, oq_hbm.at[pl.ds(off, W)],
                                              so).wait()
                        pltpu.make_async_copy(bkv, okv_hbm.at[pl.ds(off, W)],
                                              so).wait()

                    gq = pltpu.async_copy(q_hbm.at[qi_v], bq, sg)
                    gkv = pltpu.async_copy(kv_hbm.at[ki_v], bkv, sg)
                    gq.wait()
                    gkv.wait()

                    @pl.when(jj + 2 < npw)
                    def _():
                        off2 = base + (jj + 2) * W
                        pltpu.async_copy(qi_hbm.at[pl.ds(off2, W)], qi_v, si)
                        pltpu.async_copy(ki_hbm.at[pl.ds(off2, W)], ki_v, si)

                    pltpu.async_copy(bq, oq_hbm.at[pl.ds(off, W)], so)
                    pltpu.async_copy(bkv, okv_hbm.at[pl.ds(off, W)], so)

        for b in (0, 1):
            pltpu.make_async_copy(bufs[b][2], oq_hbm.at[pl.ds(base, W)],
                                  bufs[b][6]).wait()
            pltpu.make_async_copy(bufs[b][3], okv_hbm.at[pl.ds(base, W)],
                                  bufs[b][6]).wait()

    return gk(qtab, kvtab, qidx, kvidx)


def _sc_scatter(msg, sidx):
    mesh = plsc.VectorSubcoreMesh(core_axis_name="c", subcore_axis_name="s")
    nblk = NE // SW // NSC

    @functools.partial(
        pl.kernel,
        out_type=jax.ShapeDtypeStruct((NSC, NPAD, DM), _f32),
        mesh=mesh,
        scratch_types=[
            pltpu.VMEM((64, DM), _f32),
            pltpu.VMEM_SHARED((NPAD, DM), _f32),
        ],
    )
    def sk(m_hbm, si_hbm, o_hbm, z_v, acc_s):
        c = lax.axis_index("c")
        s = lax.axis_index("s")

        @pl.loop(0, 64)
        def _(r):
            for cc in range(DM // 16):
                z_v[r, pl.ds(cc * 16, 16)] = jnp.zeros((16,), _f32)

        @pl.loop(0, NPAD // NSUB // 64)
        def _(z):
            pltpu.sync_copy(z_v, acc_s.at[pl.ds(s * (NPAD // NSUB) + z * 64, 64)])

        plsc.subcore_barrier()

        def body(idx_p, row_p):
            pltpu.sync_copy(row_p, acc_s.at[idx_p.at[0]], add=True)

        for half in range(NSC):
            @pl.when(c == half)
            def _():
                pltpu.emit_pipeline(
                    body,
                    grid=(nblk,),
                    in_specs=[pl.BlockSpec((1, SW),
                                           lambda i, h=half: (0, h * nblk + i)),
                              pl.BlockSpec((SW, DM),
                                           lambda i, h=half: (h * nblk + i, 0))],
                    out_specs=[],
                    core_axis_name=("s",),
                    dimension_semantics=(pltpu.PARALLEL,),
                )(si_hbm, m_hbm)

        plsc.subcore_barrier()
        r0 = s * (NPAD // NSUB)
        pltpu.sync_copy(acc_s.at[pl.ds(r0, NPAD // NSUB)],
                        o_hbm.at[c, pl.ds(r0, NPAD // NSUB)])

    return sk(msg, sidx.reshape(1, NE))


def _sc_coef(attn_flat, sidx):
    mesh = plsc.VectorSubcoreMesh(core_axis_name="c", subcore_axis_name="s")
    epw = NE // (NSC * NSUB)
    rstripe = NR // NSUB
    cp = pltpu.CompilerParams()
    if "needs_layout_passes" in pltpu.CompilerParams.__dataclass_fields__:
        cp = dataclasses.replace(cp, needs_layout_passes=False)

    @functools.partial(
        pl.kernel,
        out_type=jax.ShapeDtypeStruct((NSC, NR, DM), _f32),
        mesh=mesh,
        compiler_params=cp,
        scratch_types=[
            pltpu.VMEM((CW,), jnp.int32),
            pltpu.VMEM((CW * H + 16,), _f32),
            pltpu.VMEM((NR, DM), _f32),
            pltpu.VMEM((128,), jnp.int32),
            pltpu.VMEM_SHARED((NR, DM), _f32),
            pltpu.SemaphoreType.DMA,
            pltpu.SemaphoreType.DMA,
        ],
    )
    def ck(a_hbm, si_hbm, o_hbm, si_v, av, acc_v, ridx_v, csum_s, sem_a, sem_i):
        c = lax.axis_index("c")
        s = lax.axis_index("s")
        wid = s * NSC + c

        @pl.loop(0, NR)
        def _(r):
            for t in range(DM // 16):
                acc_v[r, pl.ds(t * 16, 16)] = jnp.zeros((16,), _f32)

        pltpu.sync_copy(acc_v.at[pl.ds(s * rstripe, rstripe)],
                        csum_s.at[pl.ds(s * rstripe, rstripe)])

        plsc.subcore_barrier()

        base = wid * epw

        @pl.loop(0, epw // CW)
        def _(j):
            b = base + j * CW

            def inner(se_i, se_a):
                ci = pltpu.async_copy(si_hbm.at[pl.ds(b, CW)], si_v, se_i)
                ca = pltpu.async_copy(a_hbm.at[pl.ds(b * H, CW * H)],
                                      av.at[pl.ds(0, CW * H)], se_a)
                ci.wait()
                ca.wait()

            pl.run_scoped(inner, pltpu.SemaphoreType.DMA, pltpu.SemaphoreType.DMA)

            @pl.loop(0, CW)
            def _(r):
                lane = jnp.arange(16, dtype=jnp.int32)
                lmask = lane < 8
                node = plsc.load_gather(si_v, [jnp.zeros((16,), jnp.int32) + r])
                fidx = node * H + lax.bitwise_and(lane, 7)
                row = lax.shift_right_logical(fidx, 7)
                col = lax.bitwise_and(fidx, 127)
                vals = av[pl.ds(r * 8, 16)]
                plsc.addupdate_scatter(acc_v, [row, col], vals, mask=lmask)

        for kblk in range(NR // 128):
            @pl.loop(0, 8)
            def _(t):
                lane = jnp.arange(16, dtype=jnp.int32)
                ridx_v[pl.ds(t * 16, 16)] = lane + (kblk * 128) + t * 16

            pltpu.sync_copy(acc_v.at[pl.ds(kblk * 128, 128)],
                            csum_s.at[ridx_v], add=True)

        plsc.subcore_barrier()
        pltpu.sync_copy(csum_s.at[pl.ds(s * rstripe, rstripe)],
                        o_hbm.at[c, pl.ds(s * rstripe, rstripe)])

    return ck(attn_flat, sidx)


def _post_body(m0_ref, m1_ref, c0_ref, c1_ref, xq_ref, exp_ref, fc_ref,
               lnp_ref, fc1_ref, fc2_ref, ln2_ref, o_ref):
    coef = jnp.dot(c0_ref[0] + c1_ref[0], exp_ref[...],
                   preferred_element_type=_f32)
    out = (m0_ref[0] + m1_ref[0]) / (coef + 1e-8)
    y = jnp.dot(out, fc_ref[...], preferred_element_type=_f32) + xq_ref[...]
    y = _ln(y, lnp_ref[0:1, :], lnp_ref[1:2, :])
    h = jnp.maximum(jnp.dot(y, fc1_ref[...], preferred_element_type=_f32), 0.0)
    z = jnp.dot(h, fc2_ref[...], preferred_element_type=_f32) + y
    o_ref[...] = _ln(z, ln2_ref[0:1, :], ln2_ref[1:2, :])


def _post_ffn(accm, coef, xq, expm, pca, pffn):
    row = lambda i: (i, 0)
    full = lambda i: (0, 0)
    lnp = jnp.stack([pca['ln_g'], pca['ln_b']])
    ln2 = jnp.stack([pffn['ln_g'], pffn['ln_b']])
    return pl.pallas_call(
        _post_body,
        grid=(NV // NB,),
        in_specs=[pl.BlockSpec((1, NB, DM), lambda i: (0, i, 0)),
                  pl.BlockSpec((1, NB, DM), lambda i: (1, i, 0)),
                  pl.BlockSpec((1, NB, H), lambda i: (0, i, 0)),
                  pl.BlockSpec((1, NB, H), lambda i: (1, i, 0)),
                  pl.BlockSpec((NB, DM), row),
                  pl.BlockSpec((H, DM), full),
                  pl.BlockSpec((DM, DM), full),
                  pl.BlockSpec((2, DM), full),
                  pl.BlockSpec((DM, DFF), full),
                  pl.BlockSpec((DFF, DM), full),
                  pl.BlockSpec((2, DM), full)],
        out_specs=pl.BlockSpec((NB, DM), row),
        out_shape=jax.ShapeDtypeStruct((NV, DM), _f32),
    )(accm, accm, coef, coef, xq, expm, pca['fc'], lnp,
      pffn['fc1'], pffn['fc2'], ln2)


def kernel(edge_indices, edge_features, var_features, con_features, params):
    src = edge_indices[0, 0]
    tgt = edge_indices[0, 1]
    ef = edge_features[0]
    xv = var_features[0]
    xc = con_features[0]
    p = params

    hid = np.arange(DM) // DK
    bd = jnp.asarray((hid[:, None] == hid[None, :]).astype(np.float32))
    sel = np.zeros((DM, 8), np.float32)
    for t in range(H):
        sel[t * DK:(t + 1) * DK, t] = 1.0 / DK
    sel = jnp.asarray(sel)
    expm = np.zeros((H, DM), np.float32)
    for t in range(H):
        expm[t, t * DK:(t + 1) * DK] = 1.0
    expm = jnp.asarray(expm)

    var1 = _self_attn(xv, p['sa_var'], bd)
    con1 = _self_attn(xc, p['sa_con'], bd)

    kv_v2c, q_c2v = _proj_kv_q(var1, p['ca_v2c']['WK'], p['ca_v2c']['WV'],
                               p['ca_c2v']['WQ'])
    kv_c2v, q_v2c = _proj_kv_q(con1, p['ca_c2v']['WK'], p['ca_c2v']['WV'],
                               p['ca_v2c']['WQ'])
    e_v2c, e_c2v = _proj2(ef, p['ca_v2c']['WE'], p['ca_c2v']['WE'])

    qs_v2c, kvt_v2c = _sc_gather(q_v2c, kv_v2c, src, tgt)
    msg_v2c, a8_v2c = _edge_math(qs_v2c, kvt_v2c, e_v2c, bd, sel)
    accm_v2c = _sc_scatter(msg_v2c, src)
    coef_v2c = _sc_coef(a8_v2c.reshape(NE * H), src).reshape(NSC, NPAD, H)

    qs_c2v, kvt_c2v = _sc_gather(q_c2v, kv_c2v, tgt, src)
    msg_c2v, a8_c2v = _edge_math(qs_c2v, kvt_c2v, e_c2v, bd, sel)
    accm_c2v = _sc_scatter(msg_c2v, tgt)
    coef_c2v = _sc_coef(a8_c2v.reshape(NE * H), tgt).reshape(NSC, NPAD, H)

    con_out = _post_ffn(accm_v2c, coef_v2c, con1, expm, p['ca_v2c'],
                        p['ffn_con'])
    var_out = _post_ffn(accm_c2v, coef_c2v, var1, expm, p['ca_c2v'],
                        p['ffn_var'])

    return (var_out[None], con_out[None])
